# baseline (speedup 1.0000x reference)
import functools

import jax
import jax.numpy as jnp
from jax import lax
from jax.experimental import pallas as pl
from jax.experimental.pallas import tpu as pltpu
from jax.experimental.pallas import tpu_sc as plsc

N = 10000
D = 128
H = 512
E = 320000

EPAD = 327680
EROWS = EPAD // 128
NACC = 10240
RPT = NACC // 16

_f32 = jnp.float32
_bf16 = jnp.bfloat16
_i32 = jnp.int32


@functools.lru_cache(maxsize=None)
def _sc_mesh():
    return plsc.VectorSubcoreMesh(core_axis_name="c", subcore_axis_name="s")


_RING = 2
_LOOK = 1


def _edge_loop(tbl, srcm, dstm, acc, sidx, didx, bufs, semg, sems,
               src_base, dst_base, nsteps):
    def step(g, carry):
        pltpu.sync_copy(srcm.at[pl.ds(src_base + g * 8, 8)], sidx)
        pltpu.sync_copy(dstm.at[pl.ds(dst_base + g * 8, 8)], didx)
        cg, cs = {}, {}
        for j in range(_LOOK):
            cg[j] = pltpu.async_copy(tbl.at[sidx.at[j]], bufs[j % _RING], semg)
        for j in range(8):
            jj = j + _LOOK
            if jj < 8:
                if jj - _RING >= 0:
                    cs[jj - _RING].wait()
                cg[jj] = pltpu.async_copy(tbl.at[sidx.at[jj]],
                                          bufs[jj % _RING], semg)
            cg[j].wait()
            cs[j] = pltpu.async_copy(bufs[j % _RING], acc.at[didx.at[j]],
                                     sems, add=True)
        for j in range(max(0, 8 - _RING), 8):
            cs[j].wait()
        return carry

    lax.fori_loop(0, nsteps, step, 0)


@functools.lru_cache(maxsize=None)
def _build_deg():
    @functools.partial(
        pl.kernel,
        mesh=_sc_mesh(),
        out_type=jax.ShapeDtypeStruct((2, NACC, 128), _f32),
        scratch_types=[
            pltpu.VMEM((8, 128), _i32),
            pltpu.VMEM((128, 128), _f32),
            pltpu.VMEM_SHARED((NACC, 128), _f32),
            pltpu.SemaphoreType.DMA,
        ],
    )
    def _deg(dstm, z128, ones_h, d_out, didx, onesv, dacc, semd):
        c = lax.axis_index("c")
        s = lax.axis_index("s")
        r0 = s * RPT
        pltpu.sync_copy(z128.at[pl.ds(r0, RPT)], dacc.at[pl.ds(r0, RPT)])
        pltpu.sync_copy(ones_h, onesv)
        plsc.subcore_barrier()

        base = c * (EROWS // 2) + s * (EROWS // 32)

        def step(g, carry):
            pltpu.sync_copy(dstm.at[pl.ds(base + g * 8, 8)], didx)
            css = [pltpu.async_copy(onesv, dacc.at[didx.at[j]], semd,
                                    add=True) for j in range(8)]
            for cp in css:
                cp.wait()
            return carry

        lax.fori_loop(0, EROWS // 32 // 8, step, 0)
        plsc.subcore_barrier()
        pltpu.sync_copy(dacc.at[pl.ds(r0, RPT)], d_out.at[c, pl.ds(r0, RPT)])

    return _deg


@functools.lru_cache(maxsize=None)
def _build_segsum_l0():
    @functools.partial(
        pl.kernel,
        mesh=_sc_mesh(),
        out_type=jax.ShapeDtypeStruct((2, NACC, 128), _f32),
        scratch_types=[
            pltpu.VMEM((8, 128), _i32),
            pltpu.VMEM((8, 128), _i32),
            pltpu.VMEM((128, 128), _f32),
            pltpu.VMEM((128, 128), _f32),
            pltpu.VMEM_SHARED((NACC, 128), _f32),
            pltpu.SemaphoreType.DMA,
            pltpu.SemaphoreType.DMA,
        ],
    )
    def _segsum_l0(x_hbm, srcm, dstm, z128, s_out,
                   sidx, didx, rb0, rb1, acc, semg, sems):
        c = lax.axis_index("c")
        s = lax.axis_index("s")
        r0 = s * RPT
        pltpu.sync_copy(z128.at[pl.ds(r0, RPT)], acc.at[pl.ds(r0, RPT)])
        plsc.subcore_barrier()

        base = c * (EROWS // 2) + s * (EROWS // 32)
        _edge_loop(x_hbm, srcm, dstm, acc, sidx, didx, (rb0, rb1),
                   semg, sems, base, base, EROWS // 32 // 8)
        plsc.subcore_barrier()
        pltpu.sync_copy(acc.at[pl.ds(r0, RPT)], s_out.at[c, pl.ds(r0, RPT)])

    return _segsum_l0


@functools.lru_cache(maxsize=None)
def _build_segsum_4ch():
    @functools.partial(
        pl.kernel,
        mesh=_sc_mesh(),
        out_type=jax.ShapeDtypeStruct((4, NACC, 128), _f32),
        scratch_types=[
            pltpu.VMEM((8, 128), _i32),
            pltpu.VMEM((8, 128), _i32),
            pltpu.VMEM((128, 128), _f32),
            pltpu.VMEM((128, 128), _f32),
            pltpu.VMEM_SHARED((NACC, 128), _f32),
            pltpu.SemaphoreType.DMA,
            pltpu.SemaphoreType.DMA,
        ],
    )
    def _segsum_4ch(t_flat, srcm4, dstm, z128, s_out,
                    sidx, didx, rb0, rb1, acc, semg, sems):
        c = lax.axis_index("c")
        s = lax.axis_index("s")
        r0 = s * RPT
        dst_base = s * (EROWS // 16)

        for j in range(2):
            k = c * 2 + j
            pltpu.sync_copy(z128.at[pl.ds(r0, RPT)], acc.at[pl.ds(r0, RPT)])
            plsc.subcore_barrier()
            _edge_loop(t_flat, srcm4, dstm, acc, sidx, didx,
                       (rb0, rb1), semg, sems,
                       k * EROWS + dst_base, dst_base, EROWS // 16 // 8)
            plsc.subcore_barrier()
            pltpu.sync_copy(acc.at[pl.ds(r0, RPT)], s_out.at[k, pl.ds(r0, RPT)])
            plsc.subcore_barrier()

    return _segsum_4ch


RB = 1000
GRID = N // RB


def _hw0_body(x_ref, wr_ref, b_ref, hw_ref):
    hw_ref[...] = (jnp.dot(x_ref[...], wr_ref[...], preferred_element_type=_f32)
                   + b_ref[...])


def _hw4_body(t_ref, wr_ref, b_ref, hw_ref):
    acc = jnp.zeros((RB, H), _f32)
    for c in range(4):
        acc += jnp.dot(t_ref[c], wr_ref[c], preferred_element_type=_f32)
    hw_ref[...] = acc + b_ref[...]


def _a0_body(s_ref, d_ref, hw_ref, wl_ref, pre_ref, st_ref):
    i = pl.program_id(0)
    deg = (d_ref[0, :, 0:1].astype(_f32) + d_ref[1, :, 0:1].astype(_f32))
    invd = 1.0 / jnp.maximum(deg, 1.0)
    ssum = (s_ref[0].astype(_f32) + s_ref[1].astype(_f32)) * invd
    pre = (jnp.dot(ssum, wl_ref[...], preferred_element_type=_f32)
           + hw_ref[...])
    pre_ref[...] = pre

    @pl.when(i == 0)
    def _():
        st_ref[...] = jnp.zeros((2, H), _f32)

    st_ref[...] += jnp.concatenate(
        [jnp.sum(pre, 0, keepdims=True), jnp.sum(pre * pre, 0, keepdims=True)], 0)


def _a4_body(s_ref, d_ref, hw_ref, wl_ref, pre_ref, st_ref):
    i = pl.program_id(0)
    deg = (d_ref[0, :, 0:1].astype(_f32) + d_ref[1, :, 0:1].astype(_f32))
    invd = 1.0 / jnp.maximum(deg, 1.0)
    acc = hw_ref[...]
    for c in range(4):
        acc += jnp.dot(s_ref[c].astype(_f32) * invd, wl_ref[c],
                       preferred_element_type=_f32)
    pre = acc
    pre_ref[...] = pre

    @pl.when(i == 0)
    def _():
        st_ref[...] = jnp.zeros((2, H), _f32)

    st_ref[...] += jnp.concatenate(
        [jnp.sum(pre, 0, keepdims=True), jnp.sum(pre * pre, 0, keepdims=True)], 0)


def _bn_elu(pre_ref, st_ref, g_ref, be_ref):
    mu = st_ref[0:1, :] * (1.0 / N)
    var = st_ref[1:2, :] * (1.0 / N) - mu * mu
    rs = lax.rsqrt(var + 1e-5)
    hh = (pre_ref[...] - mu) * (rs * g_ref[...]) + be_ref[...]
    return jnp.where(hh > 0, hh, jnp.exp(jnp.minimum(hh, 0.0)) - 1.0)


def _b_chunks_body(pre_ref, st_ref, g_ref, be_ref, out_ref):
    y = _bn_elu(pre_ref, st_ref, g_ref, be_ref)
    for c in range(4):
        out_ref[c] = y[:, 128 * c:128 * (c + 1)]


def _b_final_body(pre_ref, st_ref, g_ref, be_ref, out_ref):
    out_ref[...] = _bn_elu(pre_ref, st_ref, g_ref, be_ref)


def _row_spec(shape):
    nd = len(shape)
    if nd == 2:
        return pl.BlockSpec((RB, shape[1]), lambda i: (i, 0))
    return pl.BlockSpec((shape[0], RB, shape[2]), lambda i: (0, i, 0))


def _full_spec(shape):
    return pl.BlockSpec(shape, lambda i: (0,) * len(shape))


def _hw_l0(x, Wr, b):
    return pl.pallas_call(
        _hw0_body,
        grid=(GRID,),
        in_specs=[_row_spec((N, D)), _full_spec((D, H)), _full_spec((1, H))],
        out_specs=[_row_spec((N, H))],
        out_shape=[jax.ShapeDtypeStruct((N, H), _f32)],
    )(x, Wr, b.reshape(1, H))[0]


def _hw_l4(t, Wr, b):
    return pl.pallas_call(
        _hw4_body,
        grid=(GRID,),
        in_specs=[_row_spec((4, N, 128)), _full_spec((4, 128, H)),
                  _full_spec((1, H))],
        out_specs=[_row_spec((N, H))],
        out_shape=[jax.ShapeDtypeStruct((N, H), _f32)],
    )(t, Wr.reshape(4, 128, H), b.reshape(1, H))[0]


def _combine_l0(s, d, hw, Wl):
    return pl.pallas_call(
        _a0_body,
        grid=(GRID,),
        in_specs=[_row_spec((2, NACC, 128)), _row_spec((2, NACC, 128)),
                  _row_spec((N, H)), _full_spec((D, H))],
        out_specs=[_row_spec((N, H)), _full_spec((2, H))],
        out_shape=[jax.ShapeDtypeStruct((N, H), _f32),
                   jax.ShapeDtypeStruct((2, H), _f32)],
    )(s, d, hw, Wl)


def _combine_l4(s, d, hw, Wl):
    return pl.pallas_call(
        _a4_body,
        grid=(GRID,),
        in_specs=[_row_spec((4, NACC, 128)), _row_spec((2, NACC, 128)),
                  _row_spec((N, H)), _full_spec((4, 128, H))],
        out_specs=[_row_spec((N, H)), _full_spec((2, H))],
        out_shape=[jax.ShapeDtypeStruct((N, H), _f32),
                   jax.ShapeDtypeStruct((2, H), _f32)],
    )(s, d, hw, Wl.reshape(4, 128, H))


def _bn_elu_chunks(pre, st, gamma, beta):
    return pl.pallas_call(
        _b_chunks_body,
        grid=(GRID,),
        in_specs=[_row_spec((N, H)), _full_spec((2, H)),
                  _full_spec((1, H)), _full_spec((1, H))],
        out_specs=[_row_spec((4, N, 128))],
        out_shape=[jax.ShapeDtypeStruct((4, N, 128), _f32)],
    )(pre, st, gamma.reshape(1, H), beta.reshape(1, H))[0]


def _bn_elu_final(pre, st, gamma, beta):
    return pl.pallas_call(
        _b_final_body,
        grid=(GRID,),
        in_specs=[_row_spec((N, H)), _full_spec((2, H)),
                  _full_spec((1, H)), _full_spec((1, H))],
        out_specs=[_row_spec((N, H))],
        out_shape=[jax.ShapeDtypeStruct((N, H), _f32)],
    )(pre, st, gamma.reshape(1, H), beta.reshape(1, H))[0]


def kernel(x, edge_index, Wl0, Wr0, b0, gamma0, beta0,
           Wl1, Wr1, b1, gamma1, beta1, Wl2, Wr2, b2, gamma2, beta2):
    src = edge_index[0]
    dst = edge_index[1]
    pidx = jnp.arange(EPAD - E, dtype=_i32)
    srcp = jnp.concatenate([src, pidx % N])
    srcm = srcp.reshape(EROWS, 128)
    dstm = jnp.concatenate([dst, N + (pidx % 16)]).reshape(EROWS, 128)
    srcm4 = (srcp[None, :] + (jnp.arange(4, dtype=_i32) * N)[:, None]
             ).reshape(4 * EROWS, 128)
    z128 = jnp.zeros((NACC, 128), _f32)
    ones128 = jnp.ones((128, 128), _f32)

    segsum_l0 = _build_segsum_l0()
    segsum_4ch = _build_segsum_4ch()

    d = _build_deg()(dstm, z128, ones128)
    s = segsum_l0(x, srcm, dstm, z128)
    hw = _hw_l0(x, Wr0, b0)
    pre, st = _combine_l0(s, d, hw, Wl0)
    t = _bn_elu_chunks(pre, st, gamma0, beta0)

    s = segsum_4ch(t.reshape(4 * N, 128), srcm4, dstm, z128)
    hw = _hw_l4(t, Wr1, b1)
    pre, st = _combine_l4(s, d, hw, Wl1)
    t = _bn_elu_chunks(pre, st, gamma1, beta1)

    s = segsum_4ch(t.reshape(4 * N, 128), srcm4, dstm, z128)
    hw = _hw_l4(t, Wr2, b2)
    pre, st = _combine_l4(s, d, hw, Wl2)
    return _bn_elu_final(pre, st, gamma2, beta2)


# speedup vs baseline: 1.0325x; 1.0325x over previous
import functools

import jax
import jax.numpy as jnp
from jax import lax
from jax.experimental import pallas as pl
from jax.experimental.pallas import tpu as pltpu
from jax.experimental.pallas import tpu_sc as plsc

N = 10000
D = 128
H = 512
E = 320000

EPAD = 327680
EROWS = EPAD // 128
NACC = 10240
RPT = NACC // 16

_f32 = jnp.float32
_bf16 = jnp.bfloat16
_i32 = jnp.int32


@functools.lru_cache(maxsize=None)
def _sc_mesh():
    return plsc.VectorSubcoreMesh(core_axis_name="c", subcore_axis_name="s")


_RING = 2
_LOOK = 1


def _edge_loop(tbl, srcm, dstm, acc, sidx, didx, bufs, semg, sems,
               src_base, dst_base, nsteps):
    def sc_wait():
        pltpu.make_async_copy(bufs[0], acc.at[didx.at[0]], sems).wait()

    def superstep(g, first):
        pltpu.sync_copy(srcm.at[pl.ds(src_base + g * 8, 8)], sidx)
        if not first:
            for _ in range(_RING):
                sc_wait()
        pltpu.sync_copy(dstm.at[pl.ds(dst_base + g * 8, 8)], didx)
        cg = {}
        for j in range(8):
            if j >= _RING:
                sc_wait()
            cg[j] = pltpu.async_copy(tbl.at[sidx.at[j]], bufs[j % _RING], semg)
            jp = j - _LOOK
            if jp >= 0:
                cg[jp].wait()
                pltpu.async_copy(bufs[jp % _RING], acc.at[didx.at[jp]],
                                 sems, add=True)
        for j in range(8 - _LOOK, 8):
            cg[j].wait()
            pltpu.async_copy(bufs[j % _RING], acc.at[didx.at[j]], sems,
                             add=True)

    superstep(0, True)

    def step(g, carry):
        superstep(g, False)
        return carry

    lax.fori_loop(1, nsteps, step, 0)
    for _ in range(_RING):
        sc_wait()


@functools.lru_cache(maxsize=None)
def _build_deg():
    @functools.partial(
        pl.kernel,
        mesh=_sc_mesh(),
        out_type=jax.ShapeDtypeStruct((2, NACC, 128), _f32),
        scratch_types=[
            pltpu.VMEM((8, 128), _i32),
            pltpu.VMEM((128, 128), _f32),
            pltpu.VMEM_SHARED((NACC, 128), _f32),
            pltpu.SemaphoreType.DMA,
        ],
    )
    def _deg(dstm, z128, ones_h, d_out, didx, onesv, dacc, semd):
        c = lax.axis_index("c")
        s = lax.axis_index("s")
        r0 = s * RPT
        pltpu.sync_copy(z128.at[pl.ds(r0, RPT)], dacc.at[pl.ds(r0, RPT)])
        pltpu.sync_copy(ones_h, onesv)
        plsc.subcore_barrier()

        base = c * (EROWS // 2) + s * (EROWS // 32)

        def step(g, carry):
            pltpu.sync_copy(dstm.at[pl.ds(base + g * 8, 8)], didx)
            css = [pltpu.async_copy(onesv, dacc.at[didx.at[j]], semd,
                                    add=True) for j in range(8)]
            for cp in css:
                cp.wait()
            return carry

        lax.fori_loop(0, EROWS // 32 // 8, step, 0)
        plsc.subcore_barrier()
        pltpu.sync_copy(dacc.at[pl.ds(r0, RPT)], d_out.at[c, pl.ds(r0, RPT)])

    return _deg


@functools.lru_cache(maxsize=None)
def _build_segsum_l0():
    @functools.partial(
        pl.kernel,
        mesh=_sc_mesh(),
        out_type=jax.ShapeDtypeStruct((2, NACC, 128), _f32),
        scratch_types=[
            pltpu.VMEM((8, 128), _i32),
            pltpu.VMEM((8, 128), _i32),
            pltpu.VMEM((128, 128), _f32),
            pltpu.VMEM((128, 128), _f32),
            pltpu.VMEM_SHARED((NACC, 128), _f32),
            pltpu.SemaphoreType.DMA,
            pltpu.SemaphoreType.DMA,
        ],
    )
    def _segsum_l0(x_hbm, srcm, dstm, z128, s_out,
                   sidx, didx, rb0, rb1, acc, semg, sems):
        c = lax.axis_index("c")
        s = lax.axis_index("s")
        r0 = s * RPT
        pltpu.sync_copy(z128.at[pl.ds(r0, RPT)], acc.at[pl.ds(r0, RPT)])
        plsc.subcore_barrier()

        base = c * (EROWS // 2) + s * (EROWS // 32)
        _edge_loop(x_hbm, srcm, dstm, acc, sidx, didx, (rb0, rb1),
                   semg, sems, base, base, EROWS // 32 // 8)
        plsc.subcore_barrier()
        pltpu.sync_copy(acc.at[pl.ds(r0, RPT)], s_out.at[c, pl.ds(r0, RPT)])

    return _segsum_l0


@functools.lru_cache(maxsize=None)
def _build_segsum_4ch():
    @functools.partial(
        pl.kernel,
        mesh=_sc_mesh(),
        out_type=jax.ShapeDtypeStruct((4, NACC, 128), _f32),
        scratch_types=[
            pltpu.VMEM((8, 128), _i32),
            pltpu.VMEM((8, 128), _i32),
            pltpu.VMEM((128, 128), _f32),
            pltpu.VMEM((128, 128), _f32),
            pltpu.VMEM_SHARED((NACC, 128), _f32),
            pltpu.SemaphoreType.DMA,
            pltpu.SemaphoreType.DMA,
        ],
    )
    def _segsum_4ch(t_flat, srcm4, dstm, z128, s_out,
                    sidx, didx, rb0, rb1, acc, semg, sems):
        c = lax.axis_index("c")
        s = lax.axis_index("s")
        r0 = s * RPT
        dst_base = s * (EROWS // 16)

        for j in range(2):
            k = c * 2 + j
            pltpu.sync_copy(z128.at[pl.ds(r0, RPT)], acc.at[pl.ds(r0, RPT)])
            plsc.subcore_barrier()
            _edge_loop(t_flat, srcm4, dstm, acc, sidx, didx,
                       (rb0, rb1), semg, sems,
                       k * EROWS + dst_base, dst_base, EROWS // 16 // 8)
            plsc.subcore_barrier()
            pltpu.sync_copy(acc.at[pl.ds(r0, RPT)], s_out.at[k, pl.ds(r0, RPT)])
            plsc.subcore_barrier()

    return _segsum_4ch


RB = 1000
GRID = N // RB


def _hw0_body(x_ref, wr_ref, b_ref, hw_ref):
    hw_ref[...] = (jnp.dot(x_ref[...], wr_ref[...], preferred_element_type=_f32)
                   + b_ref[...])


def _hw4_body(t_ref, wr_ref, b_ref, hw_ref):
    acc = jnp.zeros((RB, H), _f32)
    for c in range(4):
        acc += jnp.dot(t_ref[c], wr_ref[c], preferred_element_type=_f32)
    hw_ref[...] = acc + b_ref[...]


def _a0_body(s_ref, d_ref, hw_ref, wl_ref, pre_ref, st_ref):
    i = pl.program_id(0)
    deg = (d_ref[0, :, 0:1].astype(_f32) + d_ref[1, :, 0:1].astype(_f32))
    invd = 1.0 / jnp.maximum(deg, 1.0)
    ssum = (s_ref[0].astype(_f32) + s_ref[1].astype(_f32)) * invd
    pre = (jnp.dot(ssum, wl_ref[...], preferred_element_type=_f32)
           + hw_ref[...])
    pre_ref[...] = pre

    @pl.when(i == 0)
    def _():
        st_ref[...] = jnp.zeros((2, H), _f32)

    st_ref[...] += jnp.concatenate(
        [jnp.sum(pre, 0, keepdims=True), jnp.sum(pre * pre, 0, keepdims=True)], 0)


def _a4_body(s_ref, d_ref, hw_ref, wl_ref, pre_ref, st_ref):
    i = pl.program_id(0)
    deg = (d_ref[0, :, 0:1].astype(_f32) + d_ref[1, :, 0:1].astype(_f32))
    invd = 1.0 / jnp.maximum(deg, 1.0)
    acc = hw_ref[...]
    for c in range(4):
        acc += jnp.dot(s_ref[c].astype(_f32) * invd, wl_ref[c],
                       preferred_element_type=_f32)
    pre = acc
    pre_ref[...] = pre

    @pl.when(i == 0)
    def _():
        st_ref[...] = jnp.zeros((2, H), _f32)

    st_ref[...] += jnp.concatenate(
        [jnp.sum(pre, 0, keepdims=True), jnp.sum(pre * pre, 0, keepdims=True)], 0)


def _bn_elu(pre_ref, st_ref, g_ref, be_ref):
    mu = st_ref[0:1, :] * (1.0 / N)
    var = st_ref[1:2, :] * (1.0 / N) - mu * mu
    rs = lax.rsqrt(var + 1e-5)
    hh = (pre_ref[...] - mu) * (rs * g_ref[...]) + be_ref[...]
    return jnp.where(hh > 0, hh, jnp.exp(jnp.minimum(hh, 0.0)) - 1.0)


def _b_chunks_body(pre_ref, st_ref, g_ref, be_ref, out_ref):
    y = _bn_elu(pre_ref, st_ref, g_ref, be_ref)
    for c in range(4):
        out_ref[c] = y[:, 128 * c:128 * (c + 1)]


def _b_final_body(pre_ref, st_ref, g_ref, be_ref, out_ref):
    out_ref[...] = _bn_elu(pre_ref, st_ref, g_ref, be_ref)


def _row_spec(shape):
    nd = len(shape)
    if nd == 2:
        return pl.BlockSpec((RB, shape[1]), lambda i: (i, 0))
    return pl.BlockSpec((shape[0], RB, shape[2]), lambda i: (0, i, 0))


def _full_spec(shape):
    return pl.BlockSpec(shape, lambda i: (0,) * len(shape))


def _hw_l0(x, Wr, b):
    return pl.pallas_call(
        _hw0_body,
        grid=(GRID,),
        in_specs=[_row_spec((N, D)), _full_spec((D, H)), _full_spec((1, H))],
        out_specs=[_row_spec((N, H))],
        out_shape=[jax.ShapeDtypeStruct((N, H), _f32)],
    )(x, Wr, b.reshape(1, H))[0]


def _hw_l4(t, Wr, b):
    return pl.pallas_call(
        _hw4_body,
        grid=(GRID,),
        in_specs=[_row_spec((4, N, 128)), _full_spec((4, 128, H)),
                  _full_spec((1, H))],
        out_specs=[_row_spec((N, H))],
        out_shape=[jax.ShapeDtypeStruct((N, H), _f32)],
    )(t, Wr.reshape(4, 128, H), b.reshape(1, H))[0]


def _combine_l0(s, d, hw, Wl):
    return pl.pallas_call(
        _a0_body,
        grid=(GRID,),
        in_specs=[_row_spec((2, NACC, 128)), _row_spec((2, NACC, 128)),
                  _row_spec((N, H)), _full_spec((D, H))],
        out_specs=[_row_spec((N, H)), _full_spec((2, H))],
        out_shape=[jax.ShapeDtypeStruct((N, H), _f32),
                   jax.ShapeDtypeStruct((2, H), _f32)],
    )(s, d, hw, Wl)


def _combine_l4(s, d, hw, Wl):
    return pl.pallas_call(
        _a4_body,
        grid=(GRID,),
        in_specs=[_row_spec((4, NACC, 128)), _row_spec((2, NACC, 128)),
                  _row_spec((N, H)), _full_spec((4, 128, H))],
        out_specs=[_row_spec((N, H)), _full_spec((2, H))],
        out_shape=[jax.ShapeDtypeStruct((N, H), _f32),
                   jax.ShapeDtypeStruct((2, H), _f32)],
    )(s, d, hw, Wl.reshape(4, 128, H))


def _bn_elu_chunks(pre, st, gamma, beta):
    return pl.pallas_call(
        _b_chunks_body,
        grid=(GRID,),
        in_specs=[_row_spec((N, H)), _full_spec((2, H)),
                  _full_spec((1, H)), _full_spec((1, H))],
        out_specs=[_row_spec((4, N, 128))],
        out_shape=[jax.ShapeDtypeStruct((4, N, 128), _f32)],
    )(pre, st, gamma.reshape(1, H), beta.reshape(1, H))[0]


def _bn_elu_final(pre, st, gamma, beta):
    return pl.pallas_call(
        _b_final_body,
        grid=(GRID,),
        in_specs=[_row_spec((N, H)), _full_spec((2, H)),
                  _full_spec((1, H)), _full_spec((1, H))],
        out_specs=[_row_spec((N, H))],
        out_shape=[jax.ShapeDtypeStruct((N, H), _f32)],
    )(pre, st, gamma.reshape(1, H), beta.reshape(1, H))[0]


def kernel(x, edge_index, Wl0, Wr0, b0, gamma0, beta0,
           Wl1, Wr1, b1, gamma1, beta1, Wl2, Wr2, b2, gamma2, beta2):
    src = edge_index[0]
    dst = edge_index[1]
    pidx = jnp.arange(EPAD - E, dtype=_i32)
    srcp = jnp.concatenate([src, pidx % N])
    srcm = srcp.reshape(EROWS, 128)
    dstm = jnp.concatenate([dst, N + (pidx % 16)]).reshape(EROWS, 128)
    srcm4 = (srcp[None, :] + (jnp.arange(4, dtype=_i32) * N)[:, None]
             ).reshape(4 * EROWS, 128)
    z128 = jnp.zeros((NACC, 128), _f32)
    ones128 = jnp.ones((128, 128), _f32)

    segsum_l0 = _build_segsum_l0()
    segsum_4ch = _build_segsum_4ch()

    d = _build_deg()(dstm, z128, ones128)
    s = segsum_l0(x, srcm, dstm, z128)
    hw = _hw_l0(x, Wr0, b0)
    pre, st = _combine_l0(s, d, hw, Wl0)
    t = _bn_elu_chunks(pre, st, gamma0, beta0)

    s = segsum_4ch(t.reshape(4 * N, 128), srcm4, dstm, z128)
    hw = _hw_l4(t, Wr1, b1)
    pre, st = _combine_l4(s, d, hw, Wl1)
    t = _bn_elu_chunks(pre, st, gamma1, beta1)

    s = segsum_4ch(t.reshape(4 * N, 128), srcm4, dstm, z128)
    hw = _hw_l4(t, Wr2, b2)
    pre, st = _combine_l4(s, d, hw, Wl2)
    return _bn_elu_final(pre, st, gamma2, beta2)
